# baseline (speedup 1.0000x reference)
import functools

import jax
import jax.numpy as jnp
from jax.experimental import pallas as pl
from jax.experimental.pallas import tpu as pltpu

N_TOK = 8192
DIM = 7168
N_EXPERTS = 256
N_GROUPS = 8
GROUP_SIZE = N_EXPERTS // N_GROUPS
TOPK_GROUPS = 4
TOPK = 8
ROUTE_SCALE = 2.5

NEG = -1e30


def _gate_kernel(x_ref, w_ref, b_ref, bcast_ref, wout_ref, iout_ref, scr_ref):
    i = pl.program_id(0)
    scores = scr_ref[(i + 1) % 2]

    scores_new = jax.lax.dot_general(
        x_ref[...], w_ref[...], (((1,), (1,)), ((), ())),
        preferred_element_type=jnp.float32,
    )

    sig = jax.nn.sigmoid(scores)
    s = sig + b_ref[...]

    br = s.shape[0]
    lane = jax.lax.broadcasted_iota(jnp.int32, (br, N_EXPERTS), 1)
    lmod = lane & (GROUP_SIZE - 1)

    m1 = s
    m2 = jnp.full_like(s, NEG)
    for d in (1, 2, 4, 8, 16):
        valid = lmod < (GROUP_SIZE - d)
        pm1 = jnp.where(valid, jnp.roll(m1, -d, axis=1), NEG)
        pm2 = jnp.where(valid, jnp.roll(m2, -d, axis=1), NEG)
        lo = jnp.minimum(m1, pm1)
        m1 = jnp.maximum(m1, pm1)
        m2 = jnp.maximum(lo, jnp.maximum(m2, pm2))
    gs = m1 + m2

    rank = jnp.zeros((br, N_EXPERTS), dtype=jnp.float32)
    for k in range(1, N_GROUPS):
        other = jnp.roll(gs, -k * GROUP_SIZE, axis=1)
        tie_wins = lane >= (N_GROUPS - k) * GROUP_SIZE
        beats = (other > gs) | ((other == gs) & tie_wins)
        rank = rank + beats.astype(jnp.float32)

    keep_sparse = jnp.where((rank < TOPK_GROUPS) & (lmod == 0), 1.0, 0.0)
    keep = jax.lax.dot_general(
        keep_sparse.astype(jnp.bfloat16), bcast_ref[...],
        (((1,), (0,)), ((), ())), preferred_element_type=jnp.float32,
    )
    cur = jnp.where(keep > 0.0, s, NEG)

    lanef = lane.astype(jnp.float32)
    w_cols = []
    i_cols = []
    for _ in range(TOPK):
        m = jnp.max(cur, axis=1, keepdims=True)
        idxf = jnp.min(jnp.where(cur == m, lanef, 1e9), axis=1, keepdims=True)
        hit = lanef == idxf
        w_cols.append(jnp.sum(jnp.where(hit, sig, 0.0), axis=1, keepdims=True))
        i_cols.append(idxf)
        cur = jnp.where(hit, NEG, cur)
    wsel = jnp.concatenate(w_cols, axis=1)
    isel = jnp.concatenate(i_cols, axis=1).astype(jnp.int32)

    wsel = wsel / jnp.sum(wsel, axis=1, keepdims=True) * ROUTE_SCALE
    wout_ref[...] = wsel
    iout_ref[...] = isel
    scr_ref[i % 2] = scores_new


@functools.partial(jax.jit, static_argnames=())
def kernel(x, weight, e_score_correction_bias):
    n = x.shape[0]
    br = 256
    bias2d = e_score_correction_bias.reshape(1, N_EXPERTS)
    jj = jax.lax.broadcasted_iota(jnp.int32, (N_EXPERTS, N_EXPERTS), 0)
    ee = jax.lax.broadcasted_iota(jnp.int32, (N_EXPERTS, N_EXPERTS), 1)
    bcast = ((jj == (ee // GROUP_SIZE) * GROUP_SIZE)).astype(jnp.bfloat16)
    nblk = n // br
    wout, iout = pl.pallas_call(
        _gate_kernel,
        grid=(nblk + 1,),
        in_specs=[
            pl.BlockSpec((br, DIM), lambda i: (jnp.minimum(i, nblk - 1), 0)),
            pl.BlockSpec((N_EXPERTS, DIM), lambda i: (0, 0)),
            pl.BlockSpec((1, N_EXPERTS), lambda i: (0, 0)),
            pl.BlockSpec((N_EXPERTS, N_EXPERTS), lambda i: (0, 0)),
        ],
        out_specs=[
            pl.BlockSpec((br, TOPK), lambda i: (jnp.maximum(i - 1, 0), 0)),
            pl.BlockSpec((br, TOPK), lambda i: (jnp.maximum(i - 1, 0), 0)),
        ],
        out_shape=[
            jax.ShapeDtypeStruct((n, TOPK), jnp.float32),
            jax.ShapeDtypeStruct((n, TOPK), jnp.int32),
        ],
        scratch_shapes=[pltpu.VMEM((2, br, N_EXPERTS), jnp.float32)],
    )(x, weight, bias2d, bcast)
    return wout, iout


# speedup vs baseline: 1.2079x; 1.2079x over previous
import functools

import jax
import jax.numpy as jnp
from jax.experimental import pallas as pl
from jax.experimental.pallas import tpu as pltpu

N_TOK = 8192
DIM = 7168
N_EXPERTS = 256
N_GROUPS = 8
GROUP_SIZE = N_EXPERTS // N_GROUPS
TOPK_GROUPS = 4
TOPK = 8
ROUTE_SCALE = 2.5

NEG = -1e30


def _gate_kernel(x_ref, w_ref, b_ref, bcast_ref, wout_ref, iout_ref, scr_ref):
    i = pl.program_id(0)
    scores = scr_ref[(i + 1) % 2]

    scores_new = jax.lax.dot_general(
        x_ref[...], w_ref[...], (((1,), (1,)), ((), ())),
        preferred_element_type=jnp.float32,
    )

    sig = jax.nn.sigmoid(scores)
    s = sig + b_ref[...]

    br = s.shape[0]
    lane = jax.lax.broadcasted_iota(jnp.int32, (br, N_EXPERTS), 1)
    lmod = lane & (GROUP_SIZE - 1)

    m1 = s
    m2 = jnp.full_like(s, NEG)
    for d in (1, 2, 4, 8, 16):
        valid = lmod < (GROUP_SIZE - d)
        pm1 = jnp.where(valid, jnp.roll(m1, -d, axis=1), NEG)
        pm2 = jnp.where(valid, jnp.roll(m2, -d, axis=1), NEG)
        lo = jnp.minimum(m1, pm1)
        m1 = jnp.maximum(m1, pm1)
        m2 = jnp.maximum(lo, jnp.maximum(m2, pm2))
    gs = m1 + m2

    rank = jnp.zeros((br, N_EXPERTS), dtype=jnp.float32)
    for k in range(1, N_GROUPS):
        other = jnp.roll(gs, -k * GROUP_SIZE, axis=1)
        tie_wins = lane >= (N_GROUPS - k) * GROUP_SIZE
        beats = (other > gs) | ((other == gs) & tie_wins)
        rank = rank + beats.astype(jnp.float32)

    keep_sparse = jnp.where((rank < TOPK_GROUPS) & (lmod == 0), 1.0, 0.0)
    keep = jax.lax.dot_general(
        keep_sparse.astype(jnp.bfloat16), bcast_ref[...],
        (((1,), (0,)), ((), ())), preferred_element_type=jnp.float32,
    )
    cur = jnp.where(keep > 0.0, s, NEG)

    lanef = lane.astype(jnp.float32)
    keyv = lanef + 0.5 * sig
    w_cols = []
    i_cols = []
    for t in range(TOPK):
        m = jnp.max(cur, axis=1, keepdims=True)
        k = jnp.min(jnp.where(cur == m, keyv, 1e9), axis=1, keepdims=True)
        idxf = jnp.floor(k)
        w_cols.append((k - idxf) * 2.0)
        i_cols.append(idxf)
        if t < TOPK - 1:
            cur = jnp.where(lanef == idxf, NEG, cur)
    wsel = jnp.concatenate(w_cols, axis=1)
    isel = jnp.concatenate(i_cols, axis=1).astype(jnp.int32)

    wsel = wsel / jnp.sum(wsel, axis=1, keepdims=True) * ROUTE_SCALE
    wout_ref[...] = wsel
    iout_ref[...] = isel
    scr_ref[i % 2] = scores_new


@functools.partial(jax.jit, static_argnames=())
def kernel(x, weight, e_score_correction_bias):
    n = x.shape[0]
    br = 512
    bias2d = e_score_correction_bias.reshape(1, N_EXPERTS)
    jj = jax.lax.broadcasted_iota(jnp.int32, (N_EXPERTS, N_EXPERTS), 0)
    ee = jax.lax.broadcasted_iota(jnp.int32, (N_EXPERTS, N_EXPERTS), 1)
    bcast = ((jj == (ee // GROUP_SIZE) * GROUP_SIZE)).astype(jnp.bfloat16)
    nblk = n // br
    wout, iout = pl.pallas_call(
        _gate_kernel,
        grid=(nblk + 1,),
        in_specs=[
            pl.BlockSpec((br, DIM), lambda i: (jnp.minimum(i, nblk - 1), 0)),
            pl.BlockSpec((N_EXPERTS, DIM), lambda i: (0, 0)),
            pl.BlockSpec((1, N_EXPERTS), lambda i: (0, 0)),
            pl.BlockSpec((N_EXPERTS, N_EXPERTS), lambda i: (0, 0)),
        ],
        out_specs=[
            pl.BlockSpec((br, TOPK), lambda i: (jnp.maximum(i - 1, 0), 0)),
            pl.BlockSpec((br, TOPK), lambda i: (jnp.maximum(i - 1, 0), 0)),
        ],
        out_shape=[
            jax.ShapeDtypeStruct((n, TOPK), jnp.float32),
            jax.ShapeDtypeStruct((n, TOPK), jnp.int32),
        ],
        scratch_shapes=[pltpu.VMEM((2, br, N_EXPERTS), jnp.float32)],
    )(x, weight, bias2d, bcast)
    return wout, iout


# speedup vs baseline: 1.2214x; 1.0112x over previous
import functools

import jax
import jax.numpy as jnp
from jax.experimental import pallas as pl
from jax.experimental.pallas import tpu as pltpu

N_TOK = 8192
DIM = 7168
N_EXPERTS = 256
N_GROUPS = 8
GROUP_SIZE = N_EXPERTS // N_GROUPS
TOPK_GROUPS = 4
TOPK = 8
ROUTE_SCALE = 2.5

NEG = -1e30


def _gate_kernel(x_ref, w_ref, b_ref, bcast_ref, wout_ref, iout_ref, scr_ref):
    i = pl.program_id(0)
    scores = scr_ref[(i + 1) % 2]

    scores_new = jax.lax.dot_general(
        x_ref[...], w_ref[...], (((1,), (1,)), ((), ())),
        preferred_element_type=jnp.float32,
    )

    sig = jax.nn.sigmoid(scores)
    s = sig + b_ref[...]

    br = s.shape[0]
    lane = jax.lax.broadcasted_iota(jnp.int32, (br, N_EXPERTS), 1)
    lmod = lane & (GROUP_SIZE - 1)

    m1 = s
    m2 = jnp.full_like(s, NEG)
    for d in (1, 2, 4, 8, 16):
        pm1 = jnp.roll(m1, -d, axis=1)
        pm2 = jnp.roll(m2, -d, axis=1)
        lo = jnp.minimum(m1, pm1)
        m1 = jnp.maximum(m1, pm1)
        m2 = jnp.maximum(lo, jnp.maximum(m2, pm2))
    gs = m1 + m2

    rank = jnp.zeros((br, N_EXPERTS), dtype=jnp.float32)
    for k in range(1, N_GROUPS // 2 + 1):
        other = jnp.roll(gs, -k * GROUP_SIZE, axis=1)
        tie_wins = lane >= (N_GROUPS - k) * GROUP_SIZE
        beats = ((other > gs) | ((other == gs) & tie_wins)).astype(jnp.float32)
        rank = rank + beats
        if k != N_GROUPS - k:
            rank = rank + (1.0 - jnp.roll(beats, k * GROUP_SIZE, axis=1))

    keep_sparse = jnp.where((rank < TOPK_GROUPS) & (lmod == 0), 1.0, 0.0)
    keep = jax.lax.dot_general(
        keep_sparse.astype(jnp.bfloat16), bcast_ref[...],
        (((1,), (0,)), ((), ())), preferred_element_type=jnp.float32,
    )
    cur = jnp.where(keep > 0.0, s, NEG)

    lanef = lane.astype(jnp.float32)
    keyv = lanef + 0.5 * sig
    w_cols = []
    i_cols = []
    for t in range(TOPK):
        m = jnp.max(cur, axis=1, keepdims=True)
        k = jnp.min(jnp.where(cur == m, keyv, 1e9), axis=1, keepdims=True)
        idxf = jnp.floor(k)
        w_cols.append((k - idxf) * 2.0)
        i_cols.append(idxf)
        if t < TOPK - 1:
            cur = jnp.where(lanef == idxf, NEG, cur)
    wsel = jnp.concatenate(w_cols, axis=1)
    isel = jnp.concatenate(i_cols, axis=1).astype(jnp.int32)

    wsel = wsel / jnp.sum(wsel, axis=1, keepdims=True) * ROUTE_SCALE
    wout_ref[...] = wsel
    iout_ref[...] = isel
    scr_ref[i % 2] = scores_new


@functools.partial(jax.jit, static_argnames=())
def kernel(x, weight, e_score_correction_bias):
    n = x.shape[0]
    br = 512
    bias2d = e_score_correction_bias.reshape(1, N_EXPERTS)
    jj = jax.lax.broadcasted_iota(jnp.int32, (N_EXPERTS, N_EXPERTS), 0)
    ee = jax.lax.broadcasted_iota(jnp.int32, (N_EXPERTS, N_EXPERTS), 1)
    bcast = ((jj == (ee // GROUP_SIZE) * GROUP_SIZE)).astype(jnp.bfloat16)
    nblk = n // br
    wout, iout = pl.pallas_call(
        _gate_kernel,
        grid=(nblk + 1,),
        in_specs=[
            pl.BlockSpec((br, DIM), lambda i: (jnp.minimum(i, nblk - 1), 0)),
            pl.BlockSpec((N_EXPERTS, DIM), lambda i: (0, 0)),
            pl.BlockSpec((1, N_EXPERTS), lambda i: (0, 0)),
            pl.BlockSpec((N_EXPERTS, N_EXPERTS), lambda i: (0, 0)),
        ],
        out_specs=[
            pl.BlockSpec((br, TOPK), lambda i: (jnp.maximum(i - 1, 0), 0)),
            pl.BlockSpec((br, TOPK), lambda i: (jnp.maximum(i - 1, 0), 0)),
        ],
        out_shape=[
            jax.ShapeDtypeStruct((n, TOPK), jnp.float32),
            jax.ShapeDtypeStruct((n, TOPK), jnp.int32),
        ],
        scratch_shapes=[pltpu.VMEM((2, br, N_EXPERTS), jnp.float32)],
    )(x, weight, bias2d, bcast)
    return wout, iout
